# baseline (speedup 1.0000x reference)
import functools

import jax
import jax.numpy as jnp
from jax.experimental import pallas as pl
from jax.experimental.pallas import tpu as pltpu

B, WIN, DM, TOPK, DEMB = 8, 128, 2048, 15, 64
NRB = 4
RB = DM // NRB

_NEG = -jnp.inf
_HI = jax.lax.Precision.HIGHEST


def _fwd(x_ref, wq_ref, bq_ref, wk_ref, bk_ref, w0_ref, b0_ref, w1_ref,
         b1_ref, out_ref, a_ref, kt_ref, deg_ref, xprev_ref):
    bp = pl.program_id(0)
    rb = pl.program_id(1)
    par = jax.lax.rem(bp, 2)
    xi = x_ref[0]

    @pl.when(bp < B)
    def _graph():
        @pl.when(rb == 0)
        def _compute_kt():
            kt_ref[...] = (
                jnp.dot(wk_ref[...].astype(jnp.bfloat16),
                        xi.astype(jnp.bfloat16),
                        preferred_element_type=jnp.float32)
                + bk_ref[...])

        row0 = rb * RB
        qt = (jnp.dot(wq_ref[...].astype(jnp.bfloat16),
                      x_ref[0, :, pl.ds(row0, RB)].astype(jnp.bfloat16),
                      preferred_element_type=jnp.float32)
              + bq_ref[...])
        adj = jax.lax.dot_general(
            qt.astype(jnp.bfloat16), kt_ref[...].astype(jnp.bfloat16),
            (((0,), (0,)), ((), ())),
            preferred_element_type=jnp.float32)

        colg = jax.lax.broadcasted_iota(jnp.int32, (RB, DM), 1)
        rowg = jax.lax.broadcasted_iota(jnp.int32, (RB, DM), 0) + row0
        adj = jnp.where(colg == rowg, _NEG, adj)

        CW = 256
        m = jnp.full((RB, 1), jnp.inf, jnp.float32)
        for _ in range((TOPK - 1) // 2):
            m1 = jnp.full((RB, CW), _NEG, jnp.float32)
            m2 = m1
            for c in range(DM // CW):
                v = adj[:, c * CW:(c + 1) * CW]
                v = jnp.where(v < m, v, _NEG)
                m2 = jnp.maximum(m2, jnp.minimum(m1, v))
                m1 = jnp.maximum(m1, v)
            t1 = jnp.max(m1, axis=1, keepdims=True)
            st = jnp.concatenate([m1, m2], axis=1)
            m = jnp.max(jnp.where(st < t1, st, _NEG), axis=1, keepdims=True)
        m = jnp.max(jnp.where(adj < m, adj, _NEG), axis=1, keepdims=True)
        ablk = jnp.where(adj >= m, 1.0, 0.0)
        a_ref[pl.ds(par * DM + row0, RB), :] = ablk.astype(jnp.bfloat16)

        @pl.when(rb == 0)
        def _init_deg():
            deg_ref[pl.ds(par, 1), :] = jnp.full((1, DM), 1.0, jnp.float32)

        deg_ref[pl.ds(par, 1), :] += jnp.sum(ablk, axis=0, keepdims=True)

        @pl.when(rb == NRB - 1)
        def _save_x():
            xprev_ref[...] = xi

    @pl.when((bp >= 1) & (rb == 0))
    def _conv():
        prev = 1 - par
        a = a_ref[pl.ds(prev * DM, DM), :]
        deg = deg_ref[pl.ds(prev, 1), :]
        dinv = 1.0 / jnp.sqrt(deg)
        dinv2 = dinv * dinv
        xp = xprev_ref[...]

        y0 = jnp.dot(w0_ref[...].astype(jnp.bfloat16),
                     xp.astype(jnp.bfloat16),
                     preferred_element_type=jnp.float32)
        z0 = (jnp.dot((y0 * dinv).astype(jnp.bfloat16), a,
                      preferred_element_type=jnp.float32)
              * dinv + y0 * dinv2 + b0_ref[...])
        h1 = jnp.maximum(z0, 0.0)
        y1 = jnp.dot(w1_ref[...].astype(jnp.bfloat16),
                     h1.astype(jnp.bfloat16),
                     preferred_element_type=jnp.float32)
        z1 = (jnp.dot((y1 * dinv).astype(jnp.bfloat16), a,
                      preferred_element_type=jnp.float32)
              * dinv + y1 * dinv2 + b1_ref[...])
        out_ref[0] = z1


@jax.jit
def kernel(x, Wq, bq, Wk, bk, W0, b0, W1, b1):
    grid = (B + 1, NRB)
    return pl.pallas_call(
        _fwd,
        grid=grid,
        in_specs=[
            pl.BlockSpec((1, WIN, DM),
                         lambda b, r: (jnp.minimum(b, B - 1), 0, 0)),
            pl.BlockSpec((DEMB, WIN), lambda b, r: (0, 0)),
            pl.BlockSpec((DEMB, 1), lambda b, r: (0, 0)),
            pl.BlockSpec((DEMB, WIN), lambda b, r: (0, 0)),
            pl.BlockSpec((DEMB, 1), lambda b, r: (0, 0)),
            pl.BlockSpec((WIN, WIN), lambda b, r: (0, 0)),
            pl.BlockSpec((WIN, 1), lambda b, r: (0, 0)),
            pl.BlockSpec((WIN, WIN), lambda b, r: (0, 0)),
            pl.BlockSpec((WIN, 1), lambda b, r: (0, 0)),
        ],
        out_specs=pl.BlockSpec((1, WIN, DM),
                               lambda b, r: (jnp.maximum(b, 1) - 1, 0, 0)),
        out_shape=jax.ShapeDtypeStruct((B, WIN, DM), jnp.float32),
        scratch_shapes=[
            pltpu.VMEM((2 * DM, DM), jnp.bfloat16),
            pltpu.VMEM((DEMB, DM), jnp.float32),
            pltpu.VMEM((2, DM), jnp.float32),
            pltpu.VMEM((WIN, DM), jnp.float32),
        ],
        compiler_params=pltpu.CompilerParams(
            dimension_semantics=("arbitrary", "arbitrary")),
    )(x, Wq, bq.reshape(DEMB, 1), Wk, bk.reshape(DEMB, 1),
      W0, b0.reshape(WIN, 1), W1, b1.reshape(WIN, 1))


# speedup vs baseline: 1.0348x; 1.0348x over previous
import functools

import jax
import jax.numpy as jnp
from jax.experimental import pallas as pl
from jax.experimental.pallas import tpu as pltpu

B, WIN, DM, TOPK, DEMB = 8, 128, 2048, 15, 64
NRB = 4
RB = DM // NRB

_NEG = -jnp.inf
_HI = jax.lax.Precision.HIGHEST


def _fwd(x_ref, wq_ref, bq_ref, wk_ref, bk_ref, w0_ref, b0_ref, w1_ref,
         b1_ref, out_ref, a_ref, kt_ref, deg_ref):
    rb = pl.program_id(1)
    xi = x_ref[0]

    @pl.when(rb == 0)
    def _compute_kt():
        kt_ref[...] = (
            jnp.dot(wk_ref[...].astype(jnp.bfloat16),
                    xi.astype(jnp.bfloat16),
                    preferred_element_type=jnp.float32)
            + bk_ref[...])

    row0 = rb * RB
    qt = (jnp.dot(wq_ref[...].astype(jnp.bfloat16),
                  x_ref[0, :, pl.ds(row0, RB)].astype(jnp.bfloat16),
                  preferred_element_type=jnp.float32)
          + bq_ref[...])
    adj = jax.lax.dot_general(
        qt.astype(jnp.bfloat16), kt_ref[...].astype(jnp.bfloat16),
        (((0,), (0,)), ((), ())),
        preferred_element_type=jnp.float32)

    colg = jax.lax.broadcasted_iota(jnp.int32, (RB, DM), 1)
    rowg = jax.lax.broadcasted_iota(jnp.int32, (RB, DM), 0) + row0
    adj = jnp.where(colg == rowg, _NEG, adj)

    CW = 256
    m = None
    for r in range((TOPK - 1) // 2):
        m1 = jnp.full((RB, CW), _NEG, jnp.float32)
        m2 = m1
        for c in range(DM // CW):
            v = adj[:, c * CW:(c + 1) * CW]
            if r > 0:
                v = jnp.where(v < m, v, _NEG)
            m2 = jnp.maximum(m2, jnp.minimum(m1, v))
            m1 = jnp.maximum(m1, v)
        t1 = jnp.max(m1, axis=1, keepdims=True)
        st = jnp.concatenate([m1, m2], axis=1)
        m = jnp.max(jnp.where(st < t1, st, _NEG), axis=1, keepdims=True)
    m = jnp.max(jnp.where(adj < m, adj, _NEG), axis=1, keepdims=True)
    ablk = jnp.where(adj >= m, 1.0, 0.0)
    a_ref[pl.ds(row0, RB), :] = ablk.astype(jnp.bfloat16)

    @pl.when(rb == 0)
    def _init_deg():
        deg_ref[...] = jnp.full((1, DM), 1.0, jnp.float32)

    deg_ref[...] += jnp.sum(ablk, axis=0, keepdims=True)

    @pl.when(rb == NRB - 1)
    def _conv():
        a = a_ref[...]
        deg = deg_ref[...]
        dinv = 1.0 / jnp.sqrt(deg)
        dinv2 = dinv * dinv

        y0 = jnp.dot(w0_ref[...].astype(jnp.bfloat16),
                     xi.astype(jnp.bfloat16),
                     preferred_element_type=jnp.float32)
        z0 = (jnp.dot((y0 * dinv).astype(jnp.bfloat16), a,
                      preferred_element_type=jnp.float32)
              * dinv + y0 * dinv2 + b0_ref[...])
        h1 = jnp.maximum(z0, 0.0)
        y1 = jnp.dot(w1_ref[...].astype(jnp.bfloat16),
                     h1.astype(jnp.bfloat16),
                     preferred_element_type=jnp.float32)
        z1 = (jnp.dot((y1 * dinv).astype(jnp.bfloat16), a,
                      preferred_element_type=jnp.float32)
              * dinv + y1 * dinv2 + b1_ref[...])
        out_ref[0] = z1


@jax.jit
def kernel(x, Wq, bq, Wk, bk, W0, b0, W1, b1):
    grid = (B, NRB)
    return pl.pallas_call(
        _fwd,
        grid=grid,
        in_specs=[
            pl.BlockSpec((1, WIN, DM), lambda b, r: (b, 0, 0)),
            pl.BlockSpec((DEMB, WIN), lambda b, r: (0, 0)),
            pl.BlockSpec((DEMB, 1), lambda b, r: (0, 0)),
            pl.BlockSpec((DEMB, WIN), lambda b, r: (0, 0)),
            pl.BlockSpec((DEMB, 1), lambda b, r: (0, 0)),
            pl.BlockSpec((WIN, WIN), lambda b, r: (0, 0)),
            pl.BlockSpec((WIN, 1), lambda b, r: (0, 0)),
            pl.BlockSpec((WIN, WIN), lambda b, r: (0, 0)),
            pl.BlockSpec((WIN, 1), lambda b, r: (0, 0)),
        ],
        out_specs=pl.BlockSpec((1, WIN, DM), lambda b, r: (b, 0, 0)),
        out_shape=jax.ShapeDtypeStruct((B, WIN, DM), jnp.float32),
        scratch_shapes=[
            pltpu.VMEM((DM, DM), jnp.bfloat16),
            pltpu.VMEM((DEMB, DM), jnp.float32),
            pltpu.VMEM((1, DM), jnp.float32),
        ],
        compiler_params=pltpu.CompilerParams(
            dimension_semantics=("parallel", "arbitrary")),
    )(x, Wq, bq.reshape(DEMB, 1), Wk, bk.reshape(DEMB, 1),
      W0, b0.reshape(WIN, 1), W1, b1.reshape(WIN, 1))


# speedup vs baseline: 1.0485x; 1.0132x over previous
import functools

import jax
import jax.numpy as jnp
from jax.experimental import pallas as pl
from jax.experimental.pallas import tpu as pltpu

B, WIN, DM, TOPK, DEMB = 8, 128, 2048, 15, 64
NRB = 4
RB = DM // NRB

_NEG = -jnp.inf
_HI = jax.lax.Precision.HIGHEST


def _fwd(x_ref, wq_ref, bq_ref, wk_ref, bk_ref, w0_ref, b0_ref, w1_ref,
         b1_ref, out_ref, a_ref, kt_ref, deg_ref):
    rb = pl.program_id(1)
    xi = x_ref[0]

    @pl.when(rb == 0)
    def _compute_kt():
        kt_ref[...] = (
            jnp.dot(wk_ref[...].astype(jnp.bfloat16),
                    xi.astype(jnp.bfloat16),
                    preferred_element_type=jnp.float32)
            + bk_ref[...])

    row0 = rb * RB
    qt = (jnp.dot(wq_ref[...].astype(jnp.bfloat16),
                  x_ref[0, :, pl.ds(row0, RB)].astype(jnp.bfloat16),
                  preferred_element_type=jnp.float32)
          + bq_ref[...])
    adj = jax.lax.dot_general(
        qt.astype(jnp.bfloat16), kt_ref[...].astype(jnp.bfloat16),
        (((0,), (0,)), ((), ())),
        preferred_element_type=jnp.float32)

    colg = jax.lax.broadcasted_iota(jnp.int32, (RB, DM), 1)
    rowg = jax.lax.broadcasted_iota(jnp.int32, (RB, DM), 0) + row0
    adj = jnp.where(colg == rowg, _NEG, adj)

    CW = 256
    m = None
    for r in range((TOPK - 1) // 2):
        m1 = jnp.full((RB, CW), _NEG, jnp.float32)
        m2 = m1
        for c in range(DM // CW):
            v = adj[:, c * CW:(c + 1) * CW]
            if r > 0:
                v = jnp.where(v < m, v, _NEG)
            m2 = jnp.maximum(m2, jnp.minimum(m1, v))
            m1 = jnp.maximum(m1, v)
        t1 = jnp.max(m1, axis=1, keepdims=True)
        u1 = jnp.max(jnp.where(m1 < t1, m1, _NEG), axis=1, keepdims=True)
        u2 = jnp.max(jnp.where(m2 < t1, m2, _NEG), axis=1, keepdims=True)
        m = jnp.maximum(u1, u2)
    m = jnp.max(jnp.where(adj < m, adj, _NEG), axis=1, keepdims=True)
    ablk = jnp.where(adj >= m, 1.0, 0.0)
    a_ref[pl.ds(row0, RB), :] = ablk.astype(jnp.bfloat16)

    @pl.when(rb == 0)
    def _init_deg():
        deg_ref[...] = jnp.full((1, DM), 1.0, jnp.float32)

    deg_ref[...] += jnp.sum(ablk, axis=0, keepdims=True)

    @pl.when(rb == NRB - 1)
    def _conv():
        a = a_ref[...]
        deg = deg_ref[...]
        dinv = 1.0 / jnp.sqrt(deg)
        dinv2 = dinv * dinv

        y0 = jnp.dot(w0_ref[...].astype(jnp.bfloat16),
                     xi.astype(jnp.bfloat16),
                     preferred_element_type=jnp.float32)
        z0 = (jnp.dot((y0 * dinv).astype(jnp.bfloat16), a,
                      preferred_element_type=jnp.float32)
              * dinv + y0 * dinv2 + b0_ref[...])
        h1 = jnp.maximum(z0, 0.0)
        y1 = jnp.dot(w1_ref[...].astype(jnp.bfloat16),
                     h1.astype(jnp.bfloat16),
                     preferred_element_type=jnp.float32)
        z1 = (jnp.dot((y1 * dinv).astype(jnp.bfloat16), a,
                      preferred_element_type=jnp.float32)
              * dinv + y1 * dinv2 + b1_ref[...])
        out_ref[0] = z1


@jax.jit
def kernel(x, Wq, bq, Wk, bk, W0, b0, W1, b1):
    grid = (B, NRB)
    return pl.pallas_call(
        _fwd,
        grid=grid,
        in_specs=[
            pl.BlockSpec((1, WIN, DM), lambda b, r: (b, 0, 0)),
            pl.BlockSpec((DEMB, WIN), lambda b, r: (0, 0)),
            pl.BlockSpec((DEMB, 1), lambda b, r: (0, 0)),
            pl.BlockSpec((DEMB, WIN), lambda b, r: (0, 0)),
            pl.BlockSpec((DEMB, 1), lambda b, r: (0, 0)),
            pl.BlockSpec((WIN, WIN), lambda b, r: (0, 0)),
            pl.BlockSpec((WIN, 1), lambda b, r: (0, 0)),
            pl.BlockSpec((WIN, WIN), lambda b, r: (0, 0)),
            pl.BlockSpec((WIN, 1), lambda b, r: (0, 0)),
        ],
        out_specs=pl.BlockSpec((1, WIN, DM), lambda b, r: (b, 0, 0)),
        out_shape=jax.ShapeDtypeStruct((B, WIN, DM), jnp.float32),
        scratch_shapes=[
            pltpu.VMEM((DM, DM), jnp.bfloat16),
            pltpu.VMEM((DEMB, DM), jnp.float32),
            pltpu.VMEM((1, DM), jnp.float32),
        ],
        compiler_params=pltpu.CompilerParams(
            dimension_semantics=("parallel", "arbitrary")),
    )(x, Wq, bq.reshape(DEMB, 1), Wk, bk.reshape(DEMB, 1),
      W0, b0.reshape(WIN, 1), W1, b1.reshape(WIN, 1))
